# baseline (speedup 1.0000x reference)
import functools

import jax
import jax.numpy as jnp
from jax import lax
from jax.experimental import pallas as pl
from jax.experimental.pallas import tpu as pltpu
from jax.experimental.pallas import tpu_sc as plsc

N = 10000
E = 64000
NUM_CORES = 2
NUM_SUBCORES = 16
LANES = 16
EPT = E // NUM_SUBCORES


def _mm2(x, Wl, Wr, bl, br, block_n):
    M, K = x.shape
    O = Wl.shape[0]
    BM = 400
    grid = (M // BM, O // block_n)

    def body(x_ref, wl_ref, wr_ref, bl_ref, br_ref, ol_ref, or_ref):
        xx = x_ref[...]
        dn = (((1,), (1,)), ((), ()))
        ol_ref[...] = lax.dot_general(
            xx, wl_ref[...], dn, preferred_element_type=jnp.float32,
            precision=lax.Precision.HIGHEST) + bl_ref[...]
        or_ref[...] = lax.dot_general(
            xx, wr_ref[...], dn, preferred_element_type=jnp.float32,
            precision=lax.Precision.HIGHEST) + br_ref[...]

    return pl.pallas_call(
        body,
        grid=grid,
        in_specs=[
            pl.BlockSpec((BM, K), lambda i, j: (i, 0)),
            pl.BlockSpec((block_n, K), lambda i, j: (j, 0)),
            pl.BlockSpec((block_n, K), lambda i, j: (j, 0)),
            pl.BlockSpec((1, block_n), lambda i, j: (0, j)),
            pl.BlockSpec((1, block_n), lambda i, j: (0, j)),
        ],
        out_specs=[pl.BlockSpec((BM, block_n), lambda i, j: (i, j))] * 2,
        out_shape=[jax.ShapeDtypeStruct((M, O), jnp.float32)] * 2,
    )(x, Wl, Wr, bl.reshape(1, O), br.reshape(1, O))


def _epilogue(nd, bias, heads, ch):
    W = heads * ch
    ROWW = nd.shape[1]
    BM = 400

    def body(nd_ref, b_ref, o_ref):
        blk = nd_ref[...]
        for h in range(heads):
            d = blk[:, W + h:W + h + 1] + 1e-16
            o_ref[:, h * ch:(h + 1) * ch] = jnp.maximum(
                blk[:, h * ch:(h + 1) * ch] / d + b_ref[:, h * ch:(h + 1) * ch],
                0.0)

    return pl.pallas_call(
        body,
        grid=(N // BM,),
        in_specs=[
            pl.BlockSpec((BM, ROWW), lambda i: (i, 0)),
            pl.BlockSpec((1, W), lambda i: (0, 0)),
        ],
        out_specs=pl.BlockSpec((BM, W), lambda i: (i, 0)),
        out_shape=jax.ShapeDtypeStruct((N, W), jnp.float32),
    )(nd, bias.reshape(1, W))


RPT = 40
DIVM = 26215
CHUNKS = 8
NPAD = CHUNKS * 32 * RPT
SEB = 2000
MYC = 2560
CLC = 512


def _make_edge_kernel(W, heads):
    ch = W // heads
    CPH = ch // LANES
    ROWW = W + LANES
    mesh = plsc.VectorSubcoreMesh(
        core_axis_name="c", subcore_axis_name="s",
        num_cores=NUM_CORES, num_subcores=NUM_SUBCORES)

    @functools.partial(
        pl.kernel,
        out_type=jax.ShapeDtypeStruct((NPAD * ROWW,), jnp.float32),
        mesh=mesh,
        scratch_types=[
            pltpu.VMEM((SEB,), jnp.int32),
            pltpu.VMEM((SEB,), jnp.int32),
            pltpu.VMEM((MYC,), jnp.int32),
            pltpu.VMEM((MYC,), jnp.int32),
            pltpu.VMEM((CLC,), jnp.int32),
            pltpu.VMEM((CLC,), jnp.int32),
            pltpu.VMEM((LANES, W), jnp.float32),
            pltpu.VMEM((LANES, W), jnp.float32),
            pltpu.VMEM((RPT, W), jnp.float32),
            pltpu.VMEM((RPT * ROWW,), jnp.float32),
            pltpu.VMEM((W,), jnp.float32),
            pltpu.SemaphoreType.DMA,
            pltpu.SemaphoreType.DMA,
        ],
        compiler_params=pltpu.CompilerParams(needs_layout_passes=False),
    )
    def ek(xl_hbm, xr_hbm, src_hbm, dst_hbm, att_hbm, out_hbm,
           sbuf, dbuf, mysrc, mydst, csrc, cdst, xlbufA, xlbufB, xrs, acc,
           attv, semA, semB):
        core = lax.axis_index("c")
        tid = lax.axis_index("s")
        lanes = lax.iota(jnp.int32, LANES)
        zidx = jnp.zeros((LANES,), jnp.int32)

        pltpu.sync_copy(att_hbm, attv)

        def meta_body(eb, mycnt):
            pltpu.sync_copy(src_hbm.at[pl.ds(eb * SEB, SEB)], sbuf)
            pltpu.sync_copy(dst_hbm.at[pl.ds(eb * SEB, SEB)], dbuf)

            def scan_body(j, cnt):
                sv = sbuf[pl.ds(j * LANES, LANES)]
                dv = dbuf[pl.ds(j * LANES, LANES)]
                blk = lax.shift_right_logical(dv * DIVM, 20)
                m = ((blk & 15) == tid) & ((lax.shift_right_logical(blk, 4) & 1) == core)
                mf = jnp.where(m, 1.0, 0.0)
                cs = plsc.cumsum(mf)
                pos = jnp.minimum(cnt + cs.astype(jnp.int32) - 1, MYC - 1)
                plsc.store_scatter(mydst, [pos], dv, mask=m)
                plsc.store_scatter(mysrc, [pos], sv, mask=m)
                return jnp.minimum(cnt + jnp.sum(mf).astype(jnp.int32), MYC)
            return lax.fori_loop(0, SEB // LANES, scan_body, mycnt)
        mycnt = lax.fori_loop(0, E // SEB, meta_body, 0)

        def chunk_body(k, carry):
            blk_id = k * 32 + core * 16 + tid
            row_lo = blk_id * RPT
            row_lo_c = jnp.minimum(row_lo, N - RPT)
            dlt = row_lo - row_lo_c
            dltv = jnp.broadcast_to(dlt, (LANES,)).astype(jnp.int32)

            cps = pltpu.async_copy(xr_hbm.at[pl.ds(row_lo_c, RPT)], xrs, semB)

            def zero_body(r, carry2):
                acc[pl.ds(r * LANES, LANES)] = jnp.zeros((LANES,), jnp.float32)
                return carry2
            lax.fori_loop(0, RPT * ROWW // LANES, zero_body, 0)

            def cscan_body(j, cnt):
                sv = mysrc[pl.ds(j * LANES, LANES)]
                dv = mydst[pl.ds(j * LANES, LANES)]
                rel = dv - row_lo
                m = ((rel >= 0) & (rel < RPT)) & ((j * LANES + lanes) < mycnt)
                mf = jnp.where(m, 1.0, 0.0)
                cs = plsc.cumsum(mf)
                pos = jnp.minimum(cnt + cs.astype(jnp.int32) - 1, CLC - 1)
                plsc.store_scatter(cdst, [pos], rel, mask=m)
                plsc.store_scatter(csrc, [pos], sv, mask=m)
                return jnp.minimum(cnt + jnp.sum(mf).astype(jnp.int32), CLC)
            nmy = lax.fori_loop(0, (mycnt + LANES - 1) // LANES, cscan_body, 0)
            nblk = (nmy + LANES - 1) // LANES
            cps.wait()

            def start_gather(bbi, xbuf, sem):
                bs = bbi * LANES
                lv2 = (lanes + bs) < nmy
                sl = jnp.where(lv2, csrc[pl.ds(bs, LANES)], 0)
                pltpu.async_copy(xl_hbm.at[sl], xbuf, sem)

            def compute_block(xbuf, bbi):
                base = bbi * LANES
                lv2 = (lanes + base) < nmy
                rowv = jnp.where(lv2, cdst[pl.ds(base, LANES)], 0)
                xrrow = rowv + dltv
                accbase = rowv * ROWW
                for h in range(heads):
                    def ag_body(g, av):
                        cbase = h * ch + g * LANES
                        attc = attv[pl.ds(cbase, LANES)]
                        for t in range(LANES):
                            cv = jnp.broadcast_to(cbase + t, (LANES,)).astype(jnp.int32)
                            xlc = plsc.load_gather(xbuf, [lanes, cv])
                            xrc = plsc.load_gather(xrs, [xrrow, cv])
                            v = xlc + xrc
                            lr = jnp.maximum(v, 0.2 * v)
                            av = av + lr * attc[t]
                        return av
                    av = lax.fori_loop(0, CPH, ag_body,
                                       jnp.zeros((LANES,), jnp.float32))
                    wv = jnp.where(lv2, jnp.exp(av), 0.0)

                    def sc_body(g, carry4):
                        cbase = h * ch + g * LANES
                        for t in range(LANES):
                            cv = jnp.broadcast_to(cbase + t, (LANES,)).astype(jnp.int32)
                            xlc = plsc.load_gather(xbuf, [lanes, cv])
                            plsc.addupdate_scatter(
                                acc, [accbase + (cbase + t)], xlc * wv)
                        return carry4
                    lax.fori_loop(0, CPH, sc_body, 0)
                    plsc.addupdate_scatter(acc, [accbase + (W + h)], wv)

            @pl.when(nblk > 0)
            def _prime():
                start_gather(0, xlbufA, semA)

            def pair_body(i, carry2):
                b0 = 2 * i
                b1 = b0 + 1

                @pl.when(b1 < nblk)
                def _sb():
                    start_gather(b1, xlbufB, semB)

                @pl.when(b0 < nblk)
                def _ca():
                    pltpu.make_async_copy(xl_hbm.at[zidx], xlbufA, semA).wait()
                    compute_block(xlbufA, b0)

                @pl.when(b0 + 2 < nblk)
                def _sa():
                    start_gather(b0 + 2, xlbufA, semA)

                @pl.when(b1 < nblk)
                def _cb():
                    pltpu.make_async_copy(xl_hbm.at[zidx], xlbufB, semB).wait()
                    compute_block(xlbufB, b1)
                return carry2
            lax.fori_loop(0, (nblk + 1) // 2, pair_body, 0)

            pltpu.sync_copy(acc, out_hbm.at[pl.ds(row_lo * ROWW, RPT * ROWW)])
            return carry
        lax.fori_loop(0, CHUNKS, chunk_body, 0)

    return ek


_edge_l1 = _make_edge_kernel(W=1024, heads=4)
_edge_l2 = _make_edge_kernel(W=256, heads=1)


def kernel(node_features, edge_index, Wl1, bl1, Wr1, br1, att1, bias1,
           Wl2, bl2, Wr2, br2, att2, bias2):
    src = edge_index[0]
    dst = edge_index[1]

    xl1, xr1 = _mm2(node_features, Wl1, Wr1, bl1, br1, block_n=512)
    nd1 = _edge_l1(xl1, xr1, src, dst, att1.reshape(-1)).reshape(NPAD, 1024 + LANES)
    h = _epilogue(nd1, bias1, heads=4, ch=256)

    xl2, xr2 = _mm2(h, Wl2, Wr2, bl2, br2, block_n=256)
    nd2 = _edge_l2(xl2, xr2, src, dst, att2.reshape(-1)).reshape(NPAD, 256 + LANES)
    out = _epilogue(nd2, bias2, heads=1, ch=256)
    return out


# speedup vs baseline: 3.1646x; 3.1646x over previous
import functools

import jax
import jax.numpy as jnp
from jax import lax
from jax.experimental import pallas as pl
from jax.experimental.pallas import tpu as pltpu
from jax.experimental.pallas import tpu_sc as plsc

N = 10000
E = 64000
NUM_CORES = 2
NUM_SUBCORES = 16
LANES = 16
EPT = E // NUM_SUBCORES


def _mm2(x, Wl, Wr, bl, br, block_n):
    M, K = x.shape
    O = Wl.shape[0]
    BM = 400
    grid = (M // BM, O // block_n)

    def body(x_ref, wl_ref, wr_ref, bl_ref, br_ref, ol_ref, or_ref):
        xx = x_ref[...]
        dn = (((1,), (1,)), ((), ()))
        ol_ref[...] = lax.dot_general(
            xx, wl_ref[...], dn, preferred_element_type=jnp.float32,
            precision=lax.Precision.HIGHEST) + bl_ref[...]
        or_ref[...] = lax.dot_general(
            xx, wr_ref[...], dn, preferred_element_type=jnp.float32,
            precision=lax.Precision.HIGHEST) + br_ref[...]

    return pl.pallas_call(
        body,
        grid=grid,
        in_specs=[
            pl.BlockSpec((BM, K), lambda i, j: (i, 0)),
            pl.BlockSpec((block_n, K), lambda i, j: (j, 0)),
            pl.BlockSpec((block_n, K), lambda i, j: (j, 0)),
            pl.BlockSpec((1, block_n), lambda i, j: (0, j)),
            pl.BlockSpec((1, block_n), lambda i, j: (0, j)),
        ],
        out_specs=[pl.BlockSpec((BM, block_n), lambda i, j: (i, j))] * 2,
        out_shape=[jax.ShapeDtypeStruct((M, O), jnp.float32)] * 2,
    )(x, Wl, Wr, bl.reshape(1, O), br.reshape(1, O))


def _epilogue(nd, bias, heads, ch):
    W = heads * ch
    ROWW = nd.shape[1]
    BM = 400

    def body(nd_ref, b_ref, o_ref):
        blk = nd_ref[...]
        for h in range(heads):
            d = blk[:, W + h:W + h + 1] + 1e-16
            o_ref[:, h * ch:(h + 1) * ch] = jnp.maximum(
                blk[:, h * ch:(h + 1) * ch] / d + b_ref[:, h * ch:(h + 1) * ch],
                0.0)

    return pl.pallas_call(
        body,
        grid=(N // BM,),
        in_specs=[
            pl.BlockSpec((BM, ROWW), lambda i: (i, 0)),
            pl.BlockSpec((1, W), lambda i: (0, 0)),
        ],
        out_specs=pl.BlockSpec((BM, W), lambda i: (i, 0)),
        out_shape=jax.ShapeDtypeStruct((N, W), jnp.float32),
    )(nd, bias.reshape(1, W))


RPT = 40
DIVM = 26215
CHUNKS = 8
NPAD = CHUNKS * 32 * RPT
SEB = 2000
MYC = 2560
CLC = 512


def _make_edge_kernel(W, heads):
    ch = W // heads
    CPH = ch // LANES
    ROWW = W + LANES
    mesh = plsc.VectorSubcoreMesh(
        core_axis_name="c", subcore_axis_name="s",
        num_cores=NUM_CORES, num_subcores=NUM_SUBCORES)

    @functools.partial(
        pl.kernel,
        out_type=jax.ShapeDtypeStruct((NPAD * ROWW,), jnp.float32),
        mesh=mesh,
        scratch_types=[
            pltpu.VMEM((SEB,), jnp.int32),
            pltpu.VMEM((SEB,), jnp.int32),
            pltpu.VMEM((MYC,), jnp.int32),
            pltpu.VMEM((MYC,), jnp.int32),
            pltpu.VMEM((CLC,), jnp.int32),
            pltpu.VMEM((CLC,), jnp.int32),
            pltpu.VMEM((LANES, W), jnp.float32),
            pltpu.VMEM((LANES, W), jnp.float32),
            pltpu.VMEM((LANES, W), jnp.float32),
            pltpu.VMEM((LANES, W), jnp.float32),
            pltpu.VMEM((RPT * ROWW,), jnp.float32),
            pltpu.VMEM((W,), jnp.float32),
            pltpu.SemaphoreType.DMA,
            pltpu.SemaphoreType.DMA,
            pltpu.SemaphoreType.DMA,
            pltpu.SemaphoreType.DMA,
        ],
        compiler_params=pltpu.CompilerParams(needs_layout_passes=False),
    )
    def ek(xl_hbm, xr_hbm, src_hbm, dst_hbm, att_hbm, out_hbm,
           sbuf, dbuf, mysrc, mydst, csrc, cdst, xlbufA, xlbufB, xrbufA,
           xrbufB, acc, attv, semA, semB, semA2, semB2):
        core = lax.axis_index("c")
        tid = lax.axis_index("s")
        lanes = lax.iota(jnp.int32, LANES)
        zidx = jnp.zeros((LANES,), jnp.int32)

        pltpu.sync_copy(att_hbm, attv)

        def meta_body(eb, mycnt):
            pltpu.sync_copy(src_hbm.at[pl.ds(eb * SEB, SEB)], sbuf)
            pltpu.sync_copy(dst_hbm.at[pl.ds(eb * SEB, SEB)], dbuf)

            def scan_body(j, cnt):
                sv = sbuf[pl.ds(j * LANES, LANES)]
                dv = dbuf[pl.ds(j * LANES, LANES)]
                blk = lax.shift_right_logical(dv * DIVM, 20)
                m = ((blk & 15) == tid) & ((lax.shift_right_logical(blk, 4) & 1) == core)
                mf = jnp.where(m, 1.0, 0.0)
                cs = plsc.cumsum(mf)
                pos = jnp.minimum(cnt + cs.astype(jnp.int32) - 1, MYC - 1)
                plsc.store_scatter(mydst, [pos], dv, mask=m)
                plsc.store_scatter(mysrc, [pos], sv, mask=m)
                return jnp.minimum(cnt + jnp.sum(mf).astype(jnp.int32), MYC)
            return lax.fori_loop(0, SEB // LANES, scan_body, mycnt)
        mycnt = lax.fori_loop(0, E // SEB, meta_body, 0)

        def chunk_body(k, carry):
            blk_id = k * 32 + core * 16 + tid
            row_lo = blk_id * RPT

            def zero_body(r, carry2):
                acc[pl.ds(r * LANES, LANES)] = jnp.zeros((LANES,), jnp.float32)
                return carry2
            lax.fori_loop(0, RPT * ROWW // LANES, zero_body, 0)

            def cscan_body(j, cnt):
                sv = mysrc[pl.ds(j * LANES, LANES)]
                dv = mydst[pl.ds(j * LANES, LANES)]
                rel = dv - row_lo
                m = ((rel >= 0) & (rel < RPT)) & ((j * LANES + lanes) < mycnt)
                mf = jnp.where(m, 1.0, 0.0)
                cs = plsc.cumsum(mf)
                pos = jnp.minimum(cnt + cs.astype(jnp.int32) - 1, CLC - 1)
                plsc.store_scatter(cdst, [pos], rel, mask=m)
                plsc.store_scatter(csrc, [pos], sv, mask=m)
                return jnp.minimum(cnt + jnp.sum(mf).astype(jnp.int32), CLC)
            nmy = lax.fori_loop(0, (mycnt + LANES - 1) // LANES, cscan_body, 0)
            nblk = (nmy + LANES - 1) // LANES

            def start_gather(bbi, xbuf, xrbuf, sem, sem2):
                bs = bbi * LANES
                lv2 = (lanes + bs) < nmy
                sl = jnp.where(lv2, csrc[pl.ds(bs, LANES)], 0)
                dg = jnp.where(lv2, cdst[pl.ds(bs, LANES)] + row_lo, 0)
                pltpu.async_copy(xl_hbm.at[sl], xbuf, sem)
                pltpu.async_copy(xr_hbm.at[dg], xrbuf, sem2)

            def compute_block(xbuf, xrbuf, bbi):
                base = bbi * LANES
                for h in range(heads):
                    attc = [attv[pl.ds(h * ch + j * LANES, LANES)]
                            for j in range(CPH)]

                    def edge_body(e, carry3):
                        @pl.when(base + e < nmy)
                        def _do():
                            ridx = jnp.broadcast_to(
                                base + e, (LANES,)).astype(jnp.int32)
                            rowv = plsc.load_gather(cdst, [ridx])
                            flatbase = rowv * ROWW + lanes
                            acc_v = jnp.zeros((LANES,), jnp.float32)
                            for j in range(CPH):
                                col = h * ch + j * LANES
                                v = (xbuf[e, pl.ds(col, LANES)]
                                     + xrbuf[e, pl.ds(col, LANES)])
                                lr = jnp.maximum(v, 0.2 * v)
                                acc_v = acc_v + lr * attc[j]
                            alpha = jnp.sum(acc_v)
                            wv = jnp.exp(jnp.broadcast_to(alpha, (LANES,)))
                            for j in range(CPH):
                                col = h * ch + j * LANES
                                plsc.addupdate_scatter(
                                    acc, [flatbase + col],
                                    xbuf[e, pl.ds(col, LANES)] * wv)
                            wtail = jnp.where(lanes == h, wv, 0.0)
                            plsc.addupdate_scatter(acc, [flatbase + W], wtail)
                        return carry3
                    lax.fori_loop(0, LANES, edge_body, 0)

            @pl.when(nblk > 0)
            def _prime():
                start_gather(0, xlbufA, xrbufA, semA, semA2)

            def pair_body(i, carry2):
                b0 = 2 * i
                b1 = b0 + 1

                @pl.when(b1 < nblk)
                def _sb():
                    start_gather(b1, xlbufB, xrbufB, semB, semB2)

                @pl.when(b0 < nblk)
                def _ca():
                    pltpu.make_async_copy(xl_hbm.at[zidx], xlbufA, semA).wait()
                    pltpu.make_async_copy(xr_hbm.at[zidx], xrbufA, semA2).wait()
                    compute_block(xlbufA, xrbufA, b0)

                @pl.when(b0 + 2 < nblk)
                def _sa():
                    start_gather(b0 + 2, xlbufA, xrbufA, semA, semA2)

                @pl.when(b1 < nblk)
                def _cb():
                    pltpu.make_async_copy(xl_hbm.at[zidx], xlbufB, semB).wait()
                    pltpu.make_async_copy(xr_hbm.at[zidx], xrbufB, semB2).wait()
                    compute_block(xlbufB, xrbufB, b1)
                return carry2
            lax.fori_loop(0, (nblk + 1) // 2, pair_body, 0)

            pltpu.sync_copy(acc, out_hbm.at[pl.ds(row_lo * ROWW, RPT * ROWW)])
            return carry
        lax.fori_loop(0, CHUNKS, chunk_body, 0)

    return ek


_edge_l1 = _make_edge_kernel(W=1024, heads=4)
_edge_l2 = _make_edge_kernel(W=256, heads=1)


def kernel(node_features, edge_index, Wl1, bl1, Wr1, br1, att1, bias1,
           Wl2, bl2, Wr2, br2, att2, bias2):
    src = edge_index[0]
    dst = edge_index[1]

    xl1, xr1 = _mm2(node_features, Wl1, Wr1, bl1, br1, block_n=512)
    nd1 = _edge_l1(xl1, xr1, src, dst, att1.reshape(-1)).reshape(NPAD, 1024 + LANES)
    h = _epilogue(nd1, bias1, heads=4, ch=256)

    xl2, xr2 = _mm2(h, Wl2, Wr2, bl2, br2, block_n=256)
    nd2 = _edge_l2(xl2, xr2, src, dst, att2.reshape(-1)).reshape(NPAD, 256 + LANES)
    out = _epilogue(nd2, bias2, heads=1, ch=256)
    return out


# speedup vs baseline: 3.2540x; 1.0282x over previous
import functools

import jax
import jax.numpy as jnp
from jax import lax
from jax.experimental import pallas as pl
from jax.experimental.pallas import tpu as pltpu
from jax.experimental.pallas import tpu_sc as plsc

N = 10000
E = 64000
NUM_CORES = 2
NUM_SUBCORES = 16
LANES = 16
EPT = E // NUM_SUBCORES


def _mm2(x, Wl, Wr, bl, br, block_n):
    M, K = x.shape
    O = Wl.shape[0]
    BM = 400
    grid = (M // BM, O // block_n)

    def body(x_ref, wl_ref, wr_ref, bl_ref, br_ref, ol_ref, or_ref):
        xx = x_ref[...]
        dn = (((1,), (1,)), ((), ()))
        ol_ref[...] = lax.dot_general(
            xx, wl_ref[...], dn, preferred_element_type=jnp.float32,
            precision=lax.Precision.HIGHEST) + bl_ref[...]
        or_ref[...] = lax.dot_general(
            xx, wr_ref[...], dn, preferred_element_type=jnp.float32,
            precision=lax.Precision.HIGHEST) + br_ref[...]

    return pl.pallas_call(
        body,
        grid=grid,
        in_specs=[
            pl.BlockSpec((BM, K), lambda i, j: (i, 0)),
            pl.BlockSpec((block_n, K), lambda i, j: (j, 0)),
            pl.BlockSpec((block_n, K), lambda i, j: (j, 0)),
            pl.BlockSpec((1, block_n), lambda i, j: (0, j)),
            pl.BlockSpec((1, block_n), lambda i, j: (0, j)),
        ],
        out_specs=[pl.BlockSpec((BM, block_n), lambda i, j: (i, j))] * 2,
        out_shape=[jax.ShapeDtypeStruct((M, O), jnp.float32)] * 2,
    )(x, Wl, Wr, bl.reshape(1, O), br.reshape(1, O))


def _epilogue(nd, bias, heads, ch):
    W = heads * ch
    ROWW = nd.shape[1]
    BM = 400

    def body(nd_ref, b_ref, o_ref):
        blk = nd_ref[...]
        for h in range(heads):
            d = blk[:, W + h:W + h + 1] + 1e-16
            o_ref[:, h * ch:(h + 1) * ch] = jnp.maximum(
                blk[:, h * ch:(h + 1) * ch] / d + b_ref[:, h * ch:(h + 1) * ch],
                0.0)

    return pl.pallas_call(
        body,
        grid=(N // BM,),
        in_specs=[
            pl.BlockSpec((BM, ROWW), lambda i: (i, 0)),
            pl.BlockSpec((1, W), lambda i: (0, 0)),
        ],
        out_specs=pl.BlockSpec((BM, W), lambda i: (i, 0)),
        out_shape=jax.ShapeDtypeStruct((N, W), jnp.float32),
    )(nd, bias.reshape(1, W))


RPT = 40
DIVM = 26215
CHUNKS = 8
NPAD = CHUNKS * 32 * RPT
SEB = 2000
MYC = 2560
CLC = 512


def _make_edge_kernel(W, heads):
    ch = W // heads
    CPH = ch // LANES
    ROWW = W + LANES
    mesh = plsc.VectorSubcoreMesh(
        core_axis_name="c", subcore_axis_name="s",
        num_cores=NUM_CORES, num_subcores=NUM_SUBCORES)

    @functools.partial(
        pl.kernel,
        out_type=jax.ShapeDtypeStruct((NPAD * ROWW,), jnp.float32),
        mesh=mesh,
        scratch_types=[
            pltpu.VMEM((SEB,), jnp.int32),
            pltpu.VMEM((SEB,), jnp.int32),
            pltpu.VMEM((MYC,), jnp.int32),
            pltpu.VMEM((MYC,), jnp.int32),
            pltpu.VMEM((CLC,), jnp.int32),
            pltpu.VMEM((CLC,), jnp.int32),
            pltpu.VMEM((LANES, W), jnp.float32),
            pltpu.VMEM((LANES, W), jnp.float32),
            pltpu.VMEM((LANES, W), jnp.float32),
            pltpu.VMEM((LANES, W), jnp.float32),
            pltpu.VMEM((RPT * ROWW,), jnp.float32),
            pltpu.VMEM((W,), jnp.float32),
            pltpu.SemaphoreType.DMA,
            pltpu.SemaphoreType.DMA,
            pltpu.SemaphoreType.DMA,
            pltpu.SemaphoreType.DMA,
        ],
        compiler_params=pltpu.CompilerParams(needs_layout_passes=False),
    )
    def ek(xl_hbm, xr_hbm, src_hbm, dst_hbm, att_hbm, out_hbm,
           sbuf, dbuf, mysrc, mydst, csrc, cdst, xlbufA, xlbufB, xrbufA,
           xrbufB, acc, attv, semA, semB, semA2, semB2):
        core = lax.axis_index("c")
        tid = lax.axis_index("s")
        lanes = lax.iota(jnp.int32, LANES)
        zidx = jnp.zeros((LANES,), jnp.int32)

        pltpu.sync_copy(att_hbm, attv)

        def meta_body(eb, mycnt):
            pltpu.sync_copy(src_hbm.at[pl.ds(eb * SEB, SEB)], sbuf)
            pltpu.sync_copy(dst_hbm.at[pl.ds(eb * SEB, SEB)], dbuf)

            def scan_body(j, cnt):
                sv = sbuf[pl.ds(j * LANES, LANES)]
                dv = dbuf[pl.ds(j * LANES, LANES)]
                blk = lax.shift_right_logical(dv * DIVM, 20)
                m = ((blk & 15) == tid) & ((lax.shift_right_logical(blk, 4) & 1) == core)
                mf = jnp.where(m, 1.0, 0.0)
                cs = plsc.cumsum(mf)
                pos = jnp.minimum(cnt + cs.astype(jnp.int32) - 1, MYC - 1)
                plsc.store_scatter(mydst, [pos], dv, mask=m)
                plsc.store_scatter(mysrc, [pos], sv, mask=m)
                return jnp.minimum(cnt + jnp.sum(mf).astype(jnp.int32), MYC)
            return lax.fori_loop(0, SEB // LANES, scan_body, mycnt)
        mycnt = lax.fori_loop(0, E // SEB, meta_body, 0)

        def chunk_body(k, carry):
            blk_id = k * 32 + core * 16 + tid
            row_lo = blk_id * RPT

            def zero_body(r, carry2):
                acc[pl.ds(r * LANES, LANES)] = jnp.zeros((LANES,), jnp.float32)
                return carry2
            lax.fori_loop(0, RPT * ROWW // LANES, zero_body, 0)

            def cscan_body(j, cnt):
                sv = mysrc[pl.ds(j * LANES, LANES)]
                dv = mydst[pl.ds(j * LANES, LANES)]
                rel = dv - row_lo
                m = ((rel >= 0) & (rel < RPT)) & ((j * LANES + lanes) < mycnt)
                mf = jnp.where(m, 1.0, 0.0)
                cs = plsc.cumsum(mf)
                pos = jnp.minimum(cnt + cs.astype(jnp.int32) - 1, CLC - 1)
                plsc.store_scatter(cdst, [pos], rel, mask=m)
                plsc.store_scatter(csrc, [pos], sv, mask=m)
                return jnp.minimum(cnt + jnp.sum(mf).astype(jnp.int32), CLC)
            nmy = lax.fori_loop(0, (mycnt + LANES - 1) // LANES, cscan_body, 0)
            nblk = (nmy + LANES - 1) // LANES

            def start_gather(bbi, xbuf, xrbuf, sem, sem2):
                bs = bbi * LANES
                lv2 = (lanes + bs) < nmy
                sl = jnp.where(lv2, csrc[pl.ds(bs, LANES)], 0)
                dg = jnp.where(lv2, cdst[pl.ds(bs, LANES)] + row_lo, 0)
                pltpu.async_copy(xl_hbm.at[sl], xbuf, sem)
                pltpu.async_copy(xr_hbm.at[dg], xrbuf, sem2)

            def compute_block(xbuf, xrbuf, bbi):
                base = bbi * LANES

                def edge_body(e, carry3):
                    @pl.when(base + e < nmy)
                    def _do():
                        ridx = jnp.broadcast_to(
                            base + e, (LANES,)).astype(jnp.int32)
                        rowv = plsc.load_gather(cdst, [ridx])
                        flatbase = rowv * ROWW + lanes
                        alpha_vec = jnp.zeros((LANES,), jnp.float32)
                        for h in range(heads):
                            parts = [jnp.zeros((LANES,), jnp.float32)
                                     for _ in range(4)]
                            for j in range(CPH):
                                col = h * ch + j * LANES
                                v = (xbuf[e, pl.ds(col, LANES)]
                                     + xrbuf[e, pl.ds(col, LANES)])
                                lr = jnp.maximum(v, 0.2 * v)
                                parts[j % 4] = (parts[j % 4]
                                                + lr * attv[pl.ds(col, LANES)])
                            alpha = jnp.sum((parts[0] + parts[1])
                                            + (parts[2] + parts[3]))
                            alpha_vec = jnp.where(
                                lanes == h,
                                jnp.broadcast_to(alpha, (LANES,)), alpha_vec)
                        wv_all = jnp.exp(alpha_vec)
                        for h in range(heads):
                            wv = jnp.broadcast_to(wv_all[h], (LANES,))
                            for j in range(CPH):
                                col = h * ch + j * LANES
                                plsc.addupdate_scatter(
                                    acc, [flatbase + col],
                                    xbuf[e, pl.ds(col, LANES)] * wv)
                        wtail = jnp.where(lanes < heads, wv_all, 0.0)
                        plsc.addupdate_scatter(acc, [flatbase + W], wtail)
                    return carry3
                lax.fori_loop(0, LANES, edge_body, 0)

            @pl.when(nblk > 0)
            def _prime():
                start_gather(0, xlbufA, xrbufA, semA, semA2)

            def pair_body(i, carry2):
                b0 = 2 * i
                b1 = b0 + 1

                @pl.when(b1 < nblk)
                def _sb():
                    start_gather(b1, xlbufB, xrbufB, semB, semB2)

                @pl.when(b0 < nblk)
                def _ca():
                    pltpu.make_async_copy(xl_hbm.at[zidx], xlbufA, semA).wait()
                    pltpu.make_async_copy(xr_hbm.at[zidx], xrbufA, semA2).wait()
                    compute_block(xlbufA, xrbufA, b0)

                @pl.when(b0 + 2 < nblk)
                def _sa():
                    start_gather(b0 + 2, xlbufA, xrbufA, semA, semA2)

                @pl.when(b1 < nblk)
                def _cb():
                    pltpu.make_async_copy(xl_hbm.at[zidx], xlbufB, semB).wait()
                    pltpu.make_async_copy(xr_hbm.at[zidx], xrbufB, semB2).wait()
                    compute_block(xlbufB, xrbufB, b1)
                return carry2
            lax.fori_loop(0, (nblk + 1) // 2, pair_body, 0)

            pltpu.sync_copy(acc, out_hbm.at[pl.ds(row_lo * ROWW, RPT * ROWW)])
            return carry
        lax.fori_loop(0, CHUNKS, chunk_body, 0)

    return ek


_edge_l1 = _make_edge_kernel(W=1024, heads=4)
_edge_l2 = _make_edge_kernel(W=256, heads=1)


def kernel(node_features, edge_index, Wl1, bl1, Wr1, br1, att1, bias1,
           Wl2, bl2, Wr2, br2, att2, bias2):
    src = edge_index[0]
    dst = edge_index[1]

    xl1, xr1 = _mm2(node_features, Wl1, Wr1, bl1, br1, block_n=512)
    nd1 = _edge_l1(xl1, xr1, src, dst, att1.reshape(-1)).reshape(NPAD, 1024 + LANES)
    h = _epilogue(nd1, bias1, heads=4, ch=256)

    xl2, xr2 = _mm2(h, Wl2, Wr2, bl2, br2, block_n=256)
    nd2 = _edge_l2(xl2, xr2, src, dst, att2.reshape(-1)).reshape(NPAD, 256 + LANES)
    out = _epilogue(nd2, bias2, heads=1, ch=256)
    return out


# speedup vs baseline: 5.5938x; 1.7191x over previous
import functools

import jax
import jax.numpy as jnp
from jax import lax
from jax.experimental import pallas as pl
from jax.experimental.pallas import tpu as pltpu
from jax.experimental.pallas import tpu_sc as plsc

N = 10000
E = 64000
NUM_CORES = 2
NUM_SUBCORES = 16
LANES = 16
EPT = E // NUM_SUBCORES


def _mm2(x, Wl, Wr, bl, br, block_n):
    M, K = x.shape
    O = Wl.shape[0]
    BM = 400
    grid = (M // BM, O // block_n)

    def body(x_ref, wl_ref, wr_ref, bl_ref, br_ref, ol_ref, or_ref):
        xx = x_ref[...]
        dn = (((1,), (1,)), ((), ()))
        ol_ref[...] = lax.dot_general(
            xx, wl_ref[...], dn, preferred_element_type=jnp.float32,
            precision=lax.Precision.HIGHEST) + bl_ref[...]
        or_ref[...] = lax.dot_general(
            xx, wr_ref[...], dn, preferred_element_type=jnp.float32,
            precision=lax.Precision.HIGHEST) + br_ref[...]

    return pl.pallas_call(
        body,
        grid=grid,
        in_specs=[
            pl.BlockSpec((BM, K), lambda i, j: (i, 0)),
            pl.BlockSpec((block_n, K), lambda i, j: (j, 0)),
            pl.BlockSpec((block_n, K), lambda i, j: (j, 0)),
            pl.BlockSpec((1, block_n), lambda i, j: (0, j)),
            pl.BlockSpec((1, block_n), lambda i, j: (0, j)),
        ],
        out_specs=[pl.BlockSpec((BM, block_n), lambda i, j: (i, j))] * 2,
        out_shape=[jax.ShapeDtypeStruct((M, O), jnp.float32)] * 2,
    )(x, Wl, Wr, bl.reshape(1, O), br.reshape(1, O))


def _epilogue(nd, bias, heads, ch):
    W = heads * ch
    ROWW = nd.shape[1]
    BM = 400

    def body(nd_ref, b_ref, o_ref):
        blk = nd_ref[...]
        for h in range(heads):
            d = blk[:, W + h:W + h + 1] + 1e-16
            o_ref[:, h * ch:(h + 1) * ch] = jnp.maximum(
                blk[:, h * ch:(h + 1) * ch] / d + b_ref[:, h * ch:(h + 1) * ch],
                0.0)

    return pl.pallas_call(
        body,
        grid=(N // BM,),
        in_specs=[
            pl.BlockSpec((BM, ROWW), lambda i: (i, 0)),
            pl.BlockSpec((1, W), lambda i: (0, 0)),
        ],
        out_specs=pl.BlockSpec((BM, W), lambda i: (i, 0)),
        out_shape=jax.ShapeDtypeStruct((N, W), jnp.float32),
    )(nd, bias.reshape(1, W))


RPT = 40
DIVM = 26215
CHUNKS = 8
NPAD = CHUNKS * 32 * RPT
SEB = 2000
MYC = 2560
CLC = 512


def _make_edge_kernel(W, heads):
    ch = W // heads
    CPH = ch // LANES
    ROWW = W + LANES
    mesh = plsc.VectorSubcoreMesh(
        core_axis_name="c", subcore_axis_name="s",
        num_cores=NUM_CORES, num_subcores=NUM_SUBCORES)

    @functools.partial(
        pl.kernel,
        out_type=jax.ShapeDtypeStruct((NPAD * ROWW,), jnp.float32),
        mesh=mesh,
        scratch_types=[
            pltpu.VMEM((SEB,), jnp.int32),
            pltpu.VMEM((SEB,), jnp.int32),
            pltpu.VMEM((MYC,), jnp.int32),
            pltpu.VMEM((MYC,), jnp.int32),
            pltpu.VMEM((CLC,), jnp.int32),
            pltpu.VMEM((CLC,), jnp.int32),
            pltpu.VMEM((LANES, W), jnp.float32),
            pltpu.VMEM((LANES, W), jnp.float32),
            pltpu.VMEM((LANES, W), jnp.float32),
            pltpu.VMEM((LANES, W), jnp.float32),
            pltpu.VMEM((RPT * ROWW,), jnp.float32),
            pltpu.VMEM((W,), jnp.float32),
            pltpu.SemaphoreType.DMA,
            pltpu.SemaphoreType.DMA,
            pltpu.SemaphoreType.DMA,
            pltpu.SemaphoreType.DMA,
        ],
        compiler_params=pltpu.CompilerParams(needs_layout_passes=False),
    )
    def ek(xl_hbm, xr_hbm, src_hbm, dst_hbm, att_hbm, out_hbm,
           sbuf, dbuf, mysrc, mydst, csrc, cdst, xlbufA, xlbufB, xrbufA,
           xrbufB, acc, attv, semA, semB, semA2, semB2):
        core = lax.axis_index("c")
        tid = lax.axis_index("s")
        lanes = lax.iota(jnp.int32, LANES)
        zidx = jnp.zeros((LANES,), jnp.int32)

        pltpu.sync_copy(att_hbm, attv)

        def meta_body(eb, mycnt):
            pltpu.sync_copy(src_hbm.at[pl.ds(eb * SEB, SEB)], sbuf)
            pltpu.sync_copy(dst_hbm.at[pl.ds(eb * SEB, SEB)], dbuf)

            def scan_body(j, cnt):
                sv = sbuf[pl.ds(j * LANES, LANES)]
                dv = dbuf[pl.ds(j * LANES, LANES)]
                blk = lax.shift_right_logical(dv * DIVM, 20)
                m = ((blk & 15) == tid) & ((lax.shift_right_logical(blk, 4) & 1) == core)
                mf = jnp.where(m, 1.0, 0.0)
                cs = plsc.cumsum(mf)
                pos = jnp.minimum(cnt + cs.astype(jnp.int32) - 1, MYC - 1)
                plsc.store_scatter(mydst, [pos], dv, mask=m)
                plsc.store_scatter(mysrc, [pos], sv, mask=m)
                return jnp.minimum(cnt + jnp.sum(mf).astype(jnp.int32), MYC)
            return lax.fori_loop(0, SEB // LANES, scan_body, mycnt)
        mycnt = lax.fori_loop(0, E // SEB, meta_body, 0)

        def chunk_body(k, carry):
            blk_id = k * 32 + core * 16 + tid
            row_lo = blk_id * RPT

            def zero_body(r, carry2):
                acc[pl.ds(r * LANES, LANES)] = jnp.zeros((LANES,), jnp.float32)
                return carry2
            lax.fori_loop(0, RPT * ROWW // LANES, zero_body, 0)

            def cscan_body(j, cnt):
                sv = mysrc[pl.ds(j * LANES, LANES)]
                dv = mydst[pl.ds(j * LANES, LANES)]
                rel = dv - row_lo
                m = ((rel >= 0) & (rel < RPT)) & ((j * LANES + lanes) < mycnt)
                mf = jnp.where(m, 1.0, 0.0)
                cs = plsc.cumsum(mf)
                pos = jnp.minimum(cnt + cs.astype(jnp.int32) - 1, CLC - 1)
                plsc.store_scatter(cdst, [pos], rel, mask=m)
                plsc.store_scatter(csrc, [pos], sv, mask=m)
                return jnp.minimum(cnt + jnp.sum(mf).astype(jnp.int32), CLC)
            nmy = lax.fori_loop(0, (mycnt + LANES - 1) // LANES, cscan_body, 0)
            nblk = (nmy + LANES - 1) // LANES

            def start_gather(bbi, xbuf, xrbuf, sem, sem2):
                bs = bbi * LANES
                lv2 = (lanes + bs) < nmy
                sl = jnp.where(lv2, csrc[pl.ds(bs, LANES)], 0)
                dg = jnp.where(lv2, cdst[pl.ds(bs, LANES)] + row_lo, 0)
                pltpu.async_copy(xl_hbm.at[sl], xbuf, sem)
                pltpu.async_copy(xr_hbm.at[dg], xrbuf, sem2)

            def compute_block(xbuf, xrbuf, bbi):
                base = bbi * LANES

                @functools.partial(plsc.parallel_loop, 0, LANES, unroll=2)
                def edge_body(e):
                    valid = (base + e) < nmy
                    ridx = jnp.broadcast_to(
                        base + e, (LANES,)).astype(jnp.int32)
                    rowv = plsc.load_gather(cdst, [ridx])
                    rowv = jnp.where(valid, rowv, 0)
                    flatbase = rowv * ROWW + lanes
                    alpha_vec = jnp.zeros((LANES,), jnp.float32)
                    for h in range(heads):
                        parts = [jnp.zeros((LANES,), jnp.float32)
                                 for _ in range(4)]
                        for j in range(CPH):
                            col = h * ch + j * LANES
                            v = (xbuf[e, pl.ds(col, LANES)]
                                 + xrbuf[e, pl.ds(col, LANES)])
                            lr = jnp.maximum(v, 0.2 * v)
                            parts[j % 4] = (parts[j % 4]
                                            + lr * attv[pl.ds(col, LANES)])
                        alpha = jnp.sum((parts[0] + parts[1])
                                        + (parts[2] + parts[3]))
                        alpha_vec = jnp.where(
                            lanes == h,
                            jnp.broadcast_to(alpha, (LANES,)), alpha_vec)
                    wv_all = jnp.exp(alpha_vec)
                    wv_all = jnp.where(valid, wv_all, 0.0)
                    for h in range(heads):
                        wv = jnp.broadcast_to(wv_all[h], (LANES,))
                        for j in range(CPH):
                            col = h * ch + j * LANES
                            plsc.addupdate_scatter(
                                acc, [flatbase + col],
                                xbuf[e, pl.ds(col, LANES)] * wv)
                    wtail = jnp.where(lanes < heads, wv_all, 0.0)
                    plsc.addupdate_scatter(acc, [flatbase + W], wtail)

            @pl.when(nblk > 0)
            def _prime():
                start_gather(0, xlbufA, xrbufA, semA, semA2)

            def pair_body(i, carry2):
                b0 = 2 * i
                b1 = b0 + 1

                @pl.when(b1 < nblk)
                def _sb():
                    start_gather(b1, xlbufB, xrbufB, semB, semB2)

                @pl.when(b0 < nblk)
                def _ca():
                    pltpu.make_async_copy(xl_hbm.at[zidx], xlbufA, semA).wait()
                    pltpu.make_async_copy(xr_hbm.at[zidx], xrbufA, semA2).wait()
                    compute_block(xlbufA, xrbufA, b0)

                @pl.when(b0 + 2 < nblk)
                def _sa():
                    start_gather(b0 + 2, xlbufA, xrbufA, semA, semA2)

                @pl.when(b1 < nblk)
                def _cb():
                    pltpu.make_async_copy(xl_hbm.at[zidx], xlbufB, semB).wait()
                    pltpu.make_async_copy(xr_hbm.at[zidx], xrbufB, semB2).wait()
                    compute_block(xlbufB, xrbufB, b1)
                return carry2
            lax.fori_loop(0, (nblk + 1) // 2, pair_body, 0)

            pltpu.sync_copy(acc, out_hbm.at[pl.ds(row_lo * ROWW, RPT * ROWW)])
            return carry
        lax.fori_loop(0, CHUNKS, chunk_body, 0)

    return ek


_edge_l1 = _make_edge_kernel(W=1024, heads=4)
_edge_l2 = _make_edge_kernel(W=256, heads=1)


def kernel(node_features, edge_index, Wl1, bl1, Wr1, br1, att1, bias1,
           Wl2, bl2, Wr2, br2, att2, bias2):
    src = edge_index[0]
    dst = edge_index[1]

    xl1, xr1 = _mm2(node_features, Wl1, Wr1, bl1, br1, block_n=512)
    nd1 = _edge_l1(xl1, xr1, src, dst, att1.reshape(-1)).reshape(NPAD, 1024 + LANES)
    h = _epilogue(nd1, bias1, heads=4, ch=256)

    xl2, xr2 = _mm2(h, Wl2, Wr2, bl2, br2, block_n=256)
    nd2 = _edge_l2(xl2, xr2, src, dst, att2.reshape(-1)).reshape(NPAD, 256 + LANES)
    out = _epilogue(nd2, bias2, heads=1, ch=256)
    return out
